# baseline (speedup 1.0000x reference)
import functools

import jax
import jax.numpy as jnp
from jax import lax
from jax.experimental import pallas as pl
from jax.experimental.pallas import tpu as pltpu
from jax.experimental.pallas import tpu_sc as plsc

BATCH = 16384
NUM_ROWS = 1000000
EMBED_DIM = 32
FEAT = 16
NUM_CORES = 2
NUM_SUBCORES = 16
NUM_WORKERS = NUM_CORES * NUM_SUBCORES
BPW = BATCH // NUM_WORKERS
LANES = 16
TC_BLOCK = 32768

S_TC = 737792
V_SC = 262144
TAIL64 = NUM_ROWS - S_TC - V_SC
TAIL_BASE = S_TC + V_SC
SC_SHARE = 8192
SUB = 2048


def _tc_project(utab_t, itab_t, wu_bcast, wi_bcast):
    def body(wu_ref, wi_ref, ut_ref, it_ref, ou_ref, oi_ref):
        ou_ref[...] = jnp.sum(ut_ref[...] * wu_ref[:, 0:1], axis=0)
        oi_ref[...] = jnp.sum(it_ref[...] * wi_ref[:, 0:1], axis=0)

    grid = pl.cdiv(S_TC, TC_BLOCK)
    return pl.pallas_call(
        body,
        grid=(grid,),
        in_specs=[
            pl.BlockSpec((EMBED_DIM, 128), lambda i: (0, 0)),
            pl.BlockSpec((EMBED_DIM, 128), lambda i: (0, 0)),
            pl.BlockSpec((EMBED_DIM, TC_BLOCK), lambda i: (0, i)),
            pl.BlockSpec((EMBED_DIM, TC_BLOCK), lambda i: (0, i)),
        ],
        out_specs=[
            pl.BlockSpec((TC_BLOCK,), lambda i: (i,)),
            pl.BlockSpec((TC_BLOCK,), lambda i: (i,)),
        ],
        out_shape=[
            jax.ShapeDtypeStruct((S_TC,), jnp.float32),
            jax.ShapeDtypeStruct((S_TC,), jnp.float32),
        ],
    )(wu_bcast, wi_bcast, utab_t, itab_t)


def _sc_stream(feat_t, utab_t, itab_t, utail1d, itail1d, w_flat, fcb16,
               gb16):
    mesh = plsc.VectorSubcoreMesh(core_axis_name="c", subcore_axis_name="s")

    @functools.partial(
        pl.kernel,
        out_type=[
            jax.ShapeDtypeStruct((BATCH,), jnp.float32),
            jax.ShapeDtypeStruct((V_SC,), jnp.float32),
            jax.ShapeDtypeStruct((V_SC,), jnp.float32),
            jax.ShapeDtypeStruct((TAIL64,), jnp.float32),
            jax.ShapeDtypeStruct((TAIL64,), jnp.float32),
        ],
        mesh=mesh,
        compiler_params=pltpu.CompilerParams(needs_layout_passes=False),
        scratch_types=[
            pltpu.VMEM((EMBED_DIM, SUB), jnp.float32),
            pltpu.VMEM((SUB,), jnp.float32),
            pltpu.VMEM((FEAT, BPW), jnp.float32),
            pltpu.VMEM((EMBED_DIM * TAIL64,), jnp.float32),
            pltpu.VMEM((TAIL64,), jnp.float32),
            pltpu.VMEM((80,), jnp.float32),
            pltpu.VMEM((LANES,), jnp.float32),
            pltpu.VMEM((LANES,), jnp.float32),
            pltpu.VMEM((BPW,), jnp.float32),
        ],
    )
    def body(feat_h, utab_h, itab_h, utail_h, itail_h, w_h, fcb_h, gb_h,
             part_h, pju_h, pji_h, ptu_h, pti_h,
             slab, projv, fslab, tailv, tailp, wv, fcbv, gbv, outv):
        wid = lax.axis_index("s") * NUM_CORES + lax.axis_index("c")
        base = pl.multiple_of(wid * BPW, BPW)
        pltpu.sync_copy(w_h, wv)
        pltpu.sync_copy(fcb_h, fcbv)
        pltpu.sync_copy(gb_h, gbv)
        pltpu.sync_copy(feat_h.at[:, pl.ds(base, BPW)], fslab)
        wfv = wv[pl.ds(64, LANES)]
        biasv = fcbv[...] + gbv[...]

        def fchunk(g, carry):
            b0 = g * LANES
            acc = biasv
            for f in range(FEAT):
                acc = acc + fslab[f, pl.ds(b0, LANES)] * wfv[f]
            outv[pl.ds(b0, LANES)] = acc
            return carry

        lax.fori_loop(0, BPW // LANES, fchunk, 0)
        pltpu.sync_copy(outv, part_h.at[pl.ds(base, BPW)])

        row0 = wid * SC_SHARE
        tstart = pl.multiple_of(S_TC + row0, 128)

        for tab_h, proj_h, tail_h, ptail_h, wlo, whi in (
            (utab_h, pju_h, utail_h, ptu_h, 0, 16),
            (itab_h, pji_h, itail_h, pti_h, 32, 48),
        ):
            w0 = wv[pl.ds(wlo, LANES)]
            w1 = wv[pl.ds(whi, LANES)]
            ws = [w0[d] for d in range(LANES)] + [w1[d] for d in range(LANES)]

            def project(width, src_off, dst_off):
                pltpu.sync_copy(
                    tab_h.at[:, pl.ds(src_off, width)],
                    slab.at[:, pl.ds(0, width)])

                def pchunk(c, carry):
                    b0 = c * LANES
                    acc = slab[0, pl.ds(b0, LANES)] * ws[0]
                    for d in range(1, EMBED_DIM):
                        acc = acc + slab[d, pl.ds(b0, LANES)] * ws[d]
                    projv[pl.ds(b0, LANES)] = acc
                    return carry

                lax.fori_loop(0, width // LANES, pchunk, 0)
                pltpu.sync_copy(
                    projv.at[pl.ds(0, width)],
                    proj_h.at[pl.ds(dst_off, width)])

            def sub_iter(s, carry):
                off = pl.multiple_of(tstart + s * SUB, 128)
                project(SUB, off, row0 + s * SUB)
                return carry

            lax.fori_loop(0, SC_SHARE // SUB, sub_iter, 0)

            @pl.when(wid == 0)
            def _():
                pltpu.sync_copy(tail_h, tailv)

                def tchunk(c, carry):
                    b0 = c * LANES
                    acc = tailv[pl.ds(b0, LANES)] * ws[0]
                    for d in range(1, EMBED_DIM):
                        acc = acc + tailv[pl.ds(d * TAIL64 + b0, LANES)] * ws[d]
                    tailp[pl.ds(b0, LANES)] = acc
                    return carry

                lax.fori_loop(0, TAIL64 // LANES, tchunk, 0)
                pltpu.sync_copy(tailp, ptail_h)

    return body(feat_t, utab_t, itab_t, utail1d, itail1d, w_flat, fcb16,
                gb16)


def _sc_combine(user, item, ptcu, ptci, pscu, psci, ptu64, pti64, partial):
    mesh = plsc.VectorSubcoreMesh(core_axis_name="c", subcore_axis_name="s")

    @functools.partial(
        pl.kernel,
        out_type=jax.ShapeDtypeStruct((BATCH,), jnp.float32),
        mesh=mesh,
        compiler_params=pltpu.CompilerParams(needs_layout_passes=False),
        scratch_types=[
            pltpu.VMEM((BPW,), jnp.int32),
            pltpu.VMEM((BPW,), jnp.int32),
            pltpu.VMEM((BPW,), jnp.int32),
            pltpu.VMEM((BPW,), jnp.int32),
            pltpu.VMEM((BPW,), jnp.float32),
            pltpu.VMEM((BPW,), jnp.float32),
            pltpu.VMEM((BPW,), jnp.float32),
            pltpu.VMEM((BPW,), jnp.float32),
            pltpu.VMEM((BPW,), jnp.float32),
            pltpu.VMEM((BPW,), jnp.float32),
            pltpu.VMEM((TAIL64,), jnp.float32),
            pltpu.VMEM((TAIL64,), jnp.float32),
            pltpu.SemaphoreType.DMA,
        ],
    )
    def body(user_h, item_h, ptcu_h, ptci_h, pscu_h, psci_h, ptu_h, pti_h,
             part_h, out_h,
             uidx, iidx, tcx, scx, gut, gus, git, gis, partv, outv,
             tpu_v, tpi_v, sem):
        wid = lax.axis_index("s") * NUM_CORES + lax.axis_index("c")
        base = pl.multiple_of(wid * BPW, BPW)
        copies = []
        for idx_ref, tc_h, sc_h, gt, gs, src_h in (
            (uidx, ptcu_h, pscu_h, gut, gus, user_h),
            (iidx, ptci_h, psci_h, git, gis, item_h),
        ):
            pltpu.sync_copy(src_h.at[pl.ds(base, BPW)], idx_ref)

            def cchunk(g, carry, idx_ref=idx_ref):
                b0 = g * LANES
                v = idx_ref[pl.ds(b0, LANES)]
                tcx[pl.ds(b0, LANES)] = jnp.minimum(v, S_TC - 1)
                scx[pl.ds(b0, LANES)] = jnp.clip(v - S_TC, 0, V_SC - 1)
                return carry

            lax.fori_loop(0, BPW // LANES, cchunk, 0)
            copies.append(pltpu.async_copy(tc_h.at[tcx], gt, sem))
            copies.append(pltpu.async_copy(sc_h.at[scx], gs, sem))
            for c in copies[-2:]:
                c.wait()
        pltpu.sync_copy(part_h.at[pl.ds(base, BPW)], partv)
        pltpu.sync_copy(ptu_h, tpu_v)
        pltpu.sync_copy(pti_h, tpi_v)

        def mchunk(g, carry):
            b0 = g * LANES
            uv = uidx[pl.ds(b0, LANES)]
            iv = iidx[pl.ds(b0, LANES)]
            ut = plsc.load_gather(
                tpu_v, [jnp.clip(uv - TAIL_BASE, 0, TAIL64 - 1)])
            it = plsc.load_gather(
                tpi_v, [jnp.clip(iv - TAIL_BASE, 0, TAIL64 - 1)])
            pu = jnp.where(uv < S_TC, gut[pl.ds(b0, LANES)],
                           jnp.where(uv < TAIL_BASE,
                                     gus[pl.ds(b0, LANES)], ut))
            pi = jnp.where(iv < S_TC, git[pl.ds(b0, LANES)],
                           jnp.where(iv < TAIL_BASE,
                                     gis[pl.ds(b0, LANES)], it))
            outv[pl.ds(b0, LANES)] = partv[pl.ds(b0, LANES)] + pu + pi
            return carry

        lax.fori_loop(0, BPW // LANES, mchunk, 0)
        pltpu.sync_copy(outv, out_h.at[pl.ds(base, BPW)])

    return body(user, item, ptcu, ptci, pscu, psci, ptu64, pti64, partial)


def kernel(user, item, item_feature, user_table, item_table, fc_w, fc_b,
           global_bias):
    w_flat = fc_w.reshape(-1)
    wu_bcast = jnp.broadcast_to(w_flat[:EMBED_DIM, None], (EMBED_DIM, 128))
    wi_bcast = jnp.broadcast_to(
        w_flat[EMBED_DIM:2 * EMBED_DIM, None], (EMBED_DIM, 128))
    fcb16 = jnp.broadcast_to(fc_b, (LANES,))
    gb16 = jnp.broadcast_to(global_bias, (LANES,))
    utab_t = user_table.T
    itab_t = item_table.T
    utail1d = lax.slice(utab_t, (0, TAIL_BASE), (EMBED_DIM, NUM_ROWS)
                        ).reshape(-1)
    itail1d = lax.slice(itab_t, (0, TAIL_BASE), (EMBED_DIM, NUM_ROWS)
                        ).reshape(-1)
    ptcu, ptci = _tc_project(utab_t, itab_t, wu_bcast, wi_bcast)
    partial, pscu, psci, ptu64, pti64 = _sc_stream(
        item_feature.T, utab_t, itab_t, utail1d, itail1d,
        w_flat, fcb16, gb16)
    out = _sc_combine(user, item, ptcu, ptci, pscu, psci, ptu64, pti64,
                      partial)
    return out.reshape(BATCH, 1)


# speedup vs baseline: 2.0900x; 2.0900x over previous
import functools

import jax
import jax.numpy as jnp
from jax import lax
from jax.experimental import pallas as pl
from jax.experimental.pallas import tpu as pltpu
from jax.experimental.pallas import tpu_sc as plsc

BATCH = 16384
NUM_ROWS = 1000000
EMBED_DIM = 32
FEAT = 16
NUM_CORES = 2
NUM_SUBCORES = 16
NUM_WORKERS = NUM_CORES * NUM_SUBCORES
BPW = BATCH // NUM_WORKERS
LANES = 16
TC_BLOCK = 40960


def _tc_project(utab_t, itab_t, wu_bcast, wi_bcast):
    def body(wu_ref, wi_ref, ut_ref, it_ref, ou_ref, oi_ref):
        ou_ref[...] = jnp.sum(ut_ref[...] * wu_ref[:, 0:1], axis=0)
        oi_ref[...] = jnp.sum(it_ref[...] * wi_ref[:, 0:1], axis=0)

    grid = pl.cdiv(NUM_ROWS, TC_BLOCK)
    return pl.pallas_call(
        body,
        grid=(grid,),
        in_specs=[
            pl.BlockSpec((EMBED_DIM, 128), lambda i: (0, 0)),
            pl.BlockSpec((EMBED_DIM, 128), lambda i: (0, 0)),
            pl.BlockSpec((EMBED_DIM, TC_BLOCK), lambda i: (0, i)),
            pl.BlockSpec((EMBED_DIM, TC_BLOCK), lambda i: (0, i)),
        ],
        out_specs=[
            pl.BlockSpec((TC_BLOCK,), lambda i: (i,)),
            pl.BlockSpec((TC_BLOCK,), lambda i: (i,)),
        ],
        out_shape=[
            jax.ShapeDtypeStruct((NUM_ROWS,), jnp.float32),
            jax.ShapeDtypeStruct((NUM_ROWS,), jnp.float32),
        ],
    )(wu_bcast, wi_bcast, utab_t, itab_t)


def _sc_combine(user, item, feat_t, proj_u, proj_i, w_flat, fcb16, gb16):
    mesh = plsc.VectorSubcoreMesh(core_axis_name="c", subcore_axis_name="s")

    @functools.partial(
        pl.kernel,
        out_type=jax.ShapeDtypeStruct((BATCH,), jnp.float32),
        mesh=mesh,
        compiler_params=pltpu.CompilerParams(needs_layout_passes=False),
        scratch_types=[
            pltpu.VMEM((BPW,), jnp.int32),
            pltpu.VMEM((BPW,), jnp.int32),
            pltpu.VMEM((BPW,), jnp.float32),
            pltpu.VMEM((BPW,), jnp.float32),
            pltpu.VMEM((FEAT, BPW), jnp.float32),
            pltpu.VMEM((80,), jnp.float32),
            pltpu.VMEM((LANES,), jnp.float32),
            pltpu.VMEM((LANES,), jnp.float32),
            pltpu.VMEM((BPW,), jnp.float32),
            pltpu.SemaphoreType.DMA,
            pltpu.SemaphoreType.DMA,
        ],
    )
    def body(user_h, item_h, feat_h, pju_h, pji_h, w_h, fcb_h, gb_h,
             out_h, uidx, iidx, pu, pi, fslab, wv, fcbv, gbv, outv,
             usem, isem):
        wid = lax.axis_index("s") * NUM_CORES + lax.axis_index("c")
        base = pl.multiple_of(wid * BPW, BPW)
        pltpu.sync_copy(user_h.at[pl.ds(base, BPW)], uidx)
        pltpu.sync_copy(item_h.at[pl.ds(base, BPW)], iidx)
        cu = pltpu.async_copy(pju_h.at[uidx], pu, usem)
        ci = pltpu.async_copy(pji_h.at[iidx], pi, isem)
        pltpu.sync_copy(feat_h.at[:, pl.ds(base, BPW)], fslab)
        pltpu.sync_copy(w_h, wv)
        pltpu.sync_copy(fcb_h, fcbv)
        pltpu.sync_copy(gb_h, gbv)
        wfv = wv[pl.ds(64, LANES)]
        biasv = fcbv[...] + gbv[...]
        cu.wait()
        ci.wait()

        def chunk(g, carry):
            b0 = g * LANES
            acc = pu[pl.ds(b0, LANES)] + pi[pl.ds(b0, LANES)] + biasv
            for f in range(FEAT):
                acc = acc + fslab[f, pl.ds(b0, LANES)] * wfv[f]
            outv[pl.ds(b0, LANES)] = acc
            return carry

        lax.fori_loop(0, BPW // LANES, chunk, 0)
        pltpu.sync_copy(outv, out_h.at[pl.ds(base, BPW)])

    return body(user, item, feat_t, proj_u, proj_i, w_flat, fcb16, gb16)


def kernel(user, item, item_feature, user_table, item_table, fc_w, fc_b,
           global_bias):
    w_flat = fc_w.reshape(-1)
    wu_bcast = jnp.broadcast_to(w_flat[:EMBED_DIM, None], (EMBED_DIM, 128))
    wi_bcast = jnp.broadcast_to(
        w_flat[EMBED_DIM:2 * EMBED_DIM, None], (EMBED_DIM, 128))
    fcb16 = jnp.broadcast_to(fc_b, (LANES,))
    gb16 = jnp.broadcast_to(global_bias, (LANES,))
    proj_u, proj_i = _tc_project(user_table.T, item_table.T,
                                 wu_bcast, wi_bcast)
    out = _sc_combine(user, item, item_feature.T, proj_u, proj_i,
                      w_flat, fcb16, gb16)
    return out.reshape(BATCH, 1)
